# baseline (speedup 1.0000x reference)
import functools

import jax
import jax.numpy as jnp
from jax import lax
from jax.experimental import pallas as pl
from jax.experimental.pallas import tpu as pltpu
from jax.experimental.pallas import tpu_sc as plsc

BATCH = 16384
DIM = 128
LANES = 16
VPR = DIM // LANES
NC = 2
NS = 16
NW = NC * NS
CHUNK = 64
B_PER_W = BATCH // NW
NCHUNK = B_PER_W // CHUNK
RING = 3


def _sc_body(idx_t_hbm, idx_c_hbm, ww_hbm, wc_hbm, dw_hbm, db_hbm, out_hbm,
             idx_t_v, idx_c_v, wbuf0, cbuf0, wbuf1, cbuf1, wbuf2, cbuf2,
             part_v, out_v, dw_v, db_v,
             sem_w0, sem_c0, sem_w1, sem_c1, sem_w2, sem_c2, sem_o):
    wid = lax.axis_index("s") * NC + lax.axis_index("c")

    h_it = pltpu.async_copy(
        idx_t_hbm.at[pl.ds(wid * B_PER_W, B_PER_W)], idx_t_v, sem_o)
    h_ic = pltpu.async_copy(
        idx_c_hbm.at[pl.ds(wid * B_PER_W, B_PER_W)], idx_c_v, sem_o)
    h_dw = pltpu.async_copy(dw_hbm, dw_v, sem_o)
    h_db = pltpu.async_copy(db_hbm, db_v, sem_o)
    h_it.wait()
    h_ic.wait()
    h_dw.wait()
    h_db.wait()

    lane = lax.iota(jnp.int32, LANES)
    zero16 = jnp.zeros((LANES,), jnp.int32)
    dw = plsc.load_gather(dw_v, [zero16, zero16])
    db = plsc.load_gather(db_v, [zero16])

    bufs = ((wbuf0, cbuf0, sem_w0, sem_c0),
            (wbuf1, cbuf1, sem_w1, sem_c1),
            (wbuf2, cbuf2, sem_w2, sem_c2))

    def fire(j):
        wb, cb, sw, sc_ = bufs[j % RING]
        hw = pltpu.async_copy(
            ww_hbm.at[idx_t_v.at[pl.ds(j * CHUNK, CHUNK)]], wb, sw)
        hc = pltpu.async_copy(
            wc_hbm.at[idx_c_v.at[pl.ds(j * CHUNK, CHUNK)]], cb, sc_)
        return hw, hc

    handles = [fire(j) for j in range(RING - 1)]
    for j in range(NCHUNK):
        if j + RING - 1 < NCHUNK:
            handles.append(fire(j + RING - 1))
        hw, hc = handles[j]
        hw.wait()
        hc.wait()
        wb, cb, _, _ = bufs[j % RING]

        def group_body(g, _, j=j, wb=wb, cb=cb):
            def row_body(r, _):
                row = g * LANES + r
                acc = wb[row, pl.ds(0, LANES)] * cb[row, pl.ds(0, LANES)]
                for i in range(1, VPR):
                    acc = acc + (wb[row, pl.ds(i * LANES, LANES)] *
                                 cb[row, pl.ds(i * LANES, LANES)])
                part_v[r, :] = acc
                return _

            lax.fori_loop(0, LANES, row_body, None, unroll=8)

            def red_body(k, acc):
                col = (lane + k) & (LANES - 1)
                return acc + plsc.load_gather(part_v, [lane, col])

            acc = lax.fori_loop(0, LANES, red_body,
                                jnp.zeros((LANES,), jnp.float32), unroll=16)
            z = acc * dw + db
            out_v[pl.ds(j * CHUNK + g * LANES, LANES)] = (
                1.0 / (1.0 + jnp.exp(-z)))
            return _

        lax.fori_loop(0, CHUNK // LANES, group_body, None)

    pltpu.async_copy(out_v, out_hbm.at[pl.ds(wid * B_PER_W, B_PER_W)],
                     sem_o).wait()


@jax.jit
def _sc_call(idx_t, idx_c, W_word, W_ctx, dense_w, dense_b):
    mesh = plsc.VectorSubcoreMesh(core_axis_name="c", subcore_axis_name="s")
    f = functools.partial(
        pl.kernel,
        mesh=mesh,
        out_type=jax.ShapeDtypeStruct((BATCH,), jnp.float32),
        compiler_params=pltpu.CompilerParams(
            needs_layout_passes=False,
            disable_bounds_checks=True,
            disable_semaphore_checks=True,
            skip_device_barrier=True,
        ),
        scratch_types=[
            pltpu.VMEM((B_PER_W,), jnp.int32),
            pltpu.VMEM((B_PER_W,), jnp.int32),
            pltpu.VMEM((CHUNK, DIM), jnp.float32),
            pltpu.VMEM((CHUNK, DIM), jnp.float32),
            pltpu.VMEM((CHUNK, DIM), jnp.float32),
            pltpu.VMEM((CHUNK, DIM), jnp.float32),
            pltpu.VMEM((CHUNK, DIM), jnp.float32),
            pltpu.VMEM((CHUNK, DIM), jnp.float32),
            pltpu.VMEM((LANES, LANES), jnp.float32),
            pltpu.VMEM((B_PER_W,), jnp.float32),
            pltpu.VMEM((1, 1), jnp.float32),
            pltpu.VMEM((1,), jnp.float32),
        ] + [pltpu.SemaphoreType.DMA] * 7,
    )(_sc_body)
    return f(idx_t, idx_c, W_word, W_ctx, dense_w, dense_b)


def kernel(word_target, word_context, W_word, W_ctx, dense_w, dense_b):
    out = _sc_call(word_target.reshape(-1), word_context.reshape(-1),
                   W_word, W_ctx, dense_w, dense_b)
    return out.reshape(BATCH, 1)


# speedup vs baseline: 1.1105x; 1.1105x over previous
import functools

import jax
import jax.numpy as jnp
from jax import lax
from jax.experimental import pallas as pl
from jax.experimental.pallas import tpu as pltpu
from jax.experimental.pallas import tpu_sc as plsc

BATCH = 16384
DIM = 128
LANES = 16
VPR = DIM // LANES
NC = 2
NS = 16
NW = NC * NS
CHUNK = 64
B_PER_W = BATCH // NW
NCHUNK = B_PER_W // CHUNK
RING = 3


def _sc_body(idx_t_hbm, idx_c_hbm, ww_hbm, wc_hbm, dw_hbm, db_hbm, out_hbm,
             idx_t_v, idx_c_v, wbuf0, cbuf0, wbuf1, cbuf1, wbuf2, cbuf2,
             out_v, dw_v, db_v,
             sem_w0, sem_c0, sem_w1, sem_c1, sem_w2, sem_c2, sem_o):
    wid = lax.axis_index("s") * NC + lax.axis_index("c")

    h_it = pltpu.async_copy(
        idx_t_hbm.at[pl.ds(wid * B_PER_W, B_PER_W)], idx_t_v, sem_o)
    h_ic = pltpu.async_copy(
        idx_c_hbm.at[pl.ds(wid * B_PER_W, B_PER_W)], idx_c_v, sem_o)
    h_dw = pltpu.async_copy(dw_hbm, dw_v, sem_o)
    h_db = pltpu.async_copy(db_hbm, db_v, sem_o)
    h_it.wait()
    h_ic.wait()
    h_dw.wait()
    h_db.wait()

    lane = lax.iota(jnp.int32, LANES)
    zero16 = jnp.zeros((LANES,), jnp.int32)
    dw = plsc.load_gather(dw_v, [zero16, zero16])
    db = plsc.load_gather(db_v, [zero16])

    bufs = ((wbuf0, cbuf0, sem_w0, sem_c0),
            (wbuf1, cbuf1, sem_w1, sem_c1),
            (wbuf2, cbuf2, sem_w2, sem_c2))

    def fire(j):
        wb, cb, sw, sc_ = bufs[j % RING]
        hw = pltpu.async_copy(
            ww_hbm.at[idx_t_v.at[pl.ds(j * CHUNK, CHUNK)]], wb, sw)
        hc = pltpu.async_copy(
            wc_hbm.at[idx_c_v.at[pl.ds(j * CHUNK, CHUNK)]], cb, sc_)
        return hw, hc

    handles = [fire(j) for j in range(RING - 1)]
    for j in range(NCHUNK):
        if j + RING - 1 < NCHUNK:
            handles.append(fire(j + RING - 1))
        hw, hc = handles[j]
        hw.wait()
        hc.wait()
        wb, cb, _, _ = bufs[j % RING]

        def group_body(g, _, j=j, wb=wb, cb=cb):
            rows = g * LANES + lane

            def col_body(k, accs):
                a0, a1, a2, a3 = accs
                c0 = (lane + 4 * k) & (DIM - 1)
                c1 = (lane + 4 * k + 1) & (DIM - 1)
                c2 = (lane + 4 * k + 2) & (DIM - 1)
                c3 = (lane + 4 * k + 3) & (DIM - 1)
                a0 = a0 + (plsc.load_gather(wb, [rows, c0]) *
                           plsc.load_gather(cb, [rows, c0]))
                a1 = a1 + (plsc.load_gather(wb, [rows, c1]) *
                           plsc.load_gather(cb, [rows, c1]))
                a2 = a2 + (plsc.load_gather(wb, [rows, c2]) *
                           plsc.load_gather(cb, [rows, c2]))
                a3 = a3 + (plsc.load_gather(wb, [rows, c3]) *
                           plsc.load_gather(cb, [rows, c3]))
                return a0, a1, a2, a3

            zv = jnp.zeros((LANES,), jnp.float32)
            a0, a1, a2, a3 = lax.fori_loop(0, DIM // 4, col_body,
                                           (zv, zv, zv, zv), unroll=8)
            acc = (a0 + a1) + (a2 + a3)
            z = acc * dw + db
            out_v[pl.ds(j * CHUNK + g * LANES, LANES)] = (
                1.0 / (1.0 + jnp.exp(-z)))
            return _

        lax.fori_loop(0, CHUNK // LANES, group_body, None)

    pltpu.async_copy(out_v, out_hbm.at[pl.ds(wid * B_PER_W, B_PER_W)],
                     sem_o).wait()


@jax.jit
def _sc_call(idx_t, idx_c, W_word, W_ctx, dense_w, dense_b):
    mesh = plsc.VectorSubcoreMesh(core_axis_name="c", subcore_axis_name="s")
    f = functools.partial(
        pl.kernel,
        mesh=mesh,
        out_type=jax.ShapeDtypeStruct((BATCH,), jnp.float32),
        compiler_params=pltpu.CompilerParams(
            needs_layout_passes=False,
            disable_bounds_checks=True,
            disable_semaphore_checks=True,
            skip_device_barrier=True,
        ),
        scratch_types=[
            pltpu.VMEM((B_PER_W,), jnp.int32),
            pltpu.VMEM((B_PER_W,), jnp.int32),
            pltpu.VMEM((CHUNK, DIM), jnp.float32),
            pltpu.VMEM((CHUNK, DIM), jnp.float32),
            pltpu.VMEM((CHUNK, DIM), jnp.float32),
            pltpu.VMEM((CHUNK, DIM), jnp.float32),
            pltpu.VMEM((CHUNK, DIM), jnp.float32),
            pltpu.VMEM((CHUNK, DIM), jnp.float32),
            pltpu.VMEM((B_PER_W,), jnp.float32),
            pltpu.VMEM((1, 1), jnp.float32),
            pltpu.VMEM((1,), jnp.float32),
        ] + [pltpu.SemaphoreType.DMA] * 7,
    )(_sc_body)
    return f(idx_t, idx_c, W_word, W_ctx, dense_w, dense_b)


def kernel(word_target, word_context, W_word, W_ctx, dense_w, dense_b):
    out = _sc_call(word_target.reshape(-1), word_context.reshape(-1),
                   W_word, W_ctx, dense_w, dense_b)
    return out.reshape(BATCH, 1)
